# baseline (speedup 1.0000x reference)
import functools
import math

import jax
import jax.numpy as jnp
from jax import lax
from jax.experimental import pallas as pl
from jax.experimental.pallas import tpu as pltpu
from jax.experimental.pallas import tpu_sc as plsc

VOCAB = 100000
SEQ_LEN = 2048
D_MODEL = 768
BATCH = 4

NUM_WORKERS = 32
POS_PER_W = SEQ_LEN // NUM_WORKERS
CHUNK = 32
NCHUNKS = BATCH * (POS_PER_W // CHUNK)
LANES = 16
GROUPS = D_MODEL // LANES
SCALE = math.sqrt(float(D_MODEL))


NBUF = 3


def _body(idx_hbm, tok_hbm, pos_hbm, out_hbm,
          i0, i1, i2, i3, i4, i5, i6, i7,
          tok_a, tok_b, tok_c, pos_v,
          sem_idx, sem_pos, sem_ga, sem_gb, sem_gc,
          sem_oa, sem_ob, sem_oc):
    wid = lax.axis_index("s") * 2 + lax.axis_index("c")
    pos_base = wid * POS_PER_W

    idx_bufs = [i0, i1, i2, i3, i4, i5, i6, i7]
    tok_bufs = [tok_a, tok_b, tok_c]
    g_sems = [sem_ga, sem_gb, sem_gc]
    o_sems = [sem_oa, sem_ob, sem_oc]

    def chunk_off(c):
        b, h = divmod(c, 2)
        return b * SEQ_LEN + pos_base + h * CHUNK

    idx_handles = [
        pltpu.async_copy(idx_hbm.at[pl.ds(chunk_off(c), CHUNK)],
                         idx_bufs[c], sem_idx)
        for c in range(NCHUNKS)
    ]
    pos_handle = pltpu.async_copy(
        pos_hbm.at[pl.ds(pos_base, POS_PER_W)], pos_v, sem_pos)
    for h in idx_handles:
        h.wait()

    g_handles = [None] * NBUF
    o_handles = [None] * NBUF
    for c in range(NBUF - 1):
        g_handles[c] = pltpu.async_copy(
            tok_hbm.at[idx_bufs[c]], tok_bufs[c], g_sems[c])

    for c in range(NCHUNKS):
        buf = c % NBUF
        nc = c + NBUF - 1
        if nc < NCHUNKS:
            nbuf = nc % NBUF
            if o_handles[nbuf] is not None:
                o_handles[nbuf].wait()
            g_handles[nbuf] = pltpu.async_copy(
                tok_hbm.at[idx_bufs[nc]], tok_bufs[nbuf], g_sems[nbuf])
        g_handles[buf].wait()
        if c == 0:
            pos_handle.wait()

        half = c % 2
        tv = tok_bufs[buf]

        @plsc.parallel_loop(0, CHUNK, unroll=2)
        def _row(r):
            for j in range(GROUPS):
                sl = pl.ds(j * LANES, LANES)
                tv[r, sl] = tv[r, sl] * SCALE + pos_v[half * CHUNK + r, sl]
        o_handles[buf] = pltpu.async_copy(
            tv, out_hbm.at[pl.ds(chunk_off(c), CHUNK)], o_sems[buf])

    for h in o_handles:
        if h is not None:
            h.wait()


@jax.jit
def _embed(idx_flat, tok_table, pos_table):
    mesh = plsc.VectorSubcoreMesh(core_axis_name="c", subcore_axis_name="s")
    k = functools.partial(
        pl.kernel,
        mesh=mesh,
        out_type=jax.ShapeDtypeStruct((BATCH * SEQ_LEN, D_MODEL), jnp.float32),
        scratch_types=[
            *[pltpu.VMEM((CHUNK,), jnp.int32) for _ in range(NCHUNKS)],
            *[pltpu.VMEM((CHUNK, D_MODEL), jnp.float32) for _ in range(NBUF)],
            pltpu.VMEM((POS_PER_W, D_MODEL), jnp.float32),
            *[pltpu.SemaphoreType.DMA for _ in range(2 + 2 * NBUF)],
        ],
    )(_body)
    return k(idx_flat, tok_table, pos_table)


def kernel(inputs, tok_table, pos_table):
    idx_flat = inputs.astype(jnp.int32).reshape(-1)
    out = _embed(idx_flat, tok_table, pos_table)
    return out.reshape(BATCH, SEQ_LEN, D_MODEL)


# speedup vs baseline: 1.0485x; 1.0485x over previous
import functools
import math

import jax
import jax.numpy as jnp
from jax import lax
from jax.experimental import pallas as pl
from jax.experimental.pallas import tpu as pltpu
from jax.experimental.pallas import tpu_sc as plsc

VOCAB = 100000
SEQ_LEN = 2048
D_MODEL = 768
BATCH = 4

NUM_WORKERS = 32
POS_PER_W = SEQ_LEN // NUM_WORKERS
CHUNK = 32
NCHUNKS = BATCH * (POS_PER_W // CHUNK)
LANES = 16
GROUPS = D_MODEL // LANES
SCALE = math.sqrt(float(D_MODEL))


NBUF = 3


def _body(idx_hbm, tok_hbm, pos_hbm, out_hbm,
          i0, i1, i2, i3, i4, i5, i6, i7,
          tok_a, tok_b, tok_c, pos_v,
          sem_idx, sem_pos, sem_ga, sem_gb, sem_gc,
          sem_oa, sem_ob, sem_oc):
    wid = lax.axis_index("s") * 2 + lax.axis_index("c")
    pos_base = wid * POS_PER_W

    idx_bufs = [i0, i1, i2, i3, i4, i5, i6, i7]
    tok_bufs = [tok_a, tok_b, tok_c]
    g_sems = [sem_ga, sem_gb, sem_gc]
    o_sems = [sem_oa, sem_ob, sem_oc]

    def chunk_off(c):
        b, h = divmod(c, 2)
        return b * SEQ_LEN + pos_base + h * CHUNK

    idx_handles = [
        pltpu.async_copy(idx_hbm.at[pl.ds(chunk_off(c), CHUNK)],
                         idx_bufs[c], sem_idx)
        for c in range(NCHUNKS)
    ]
    pos_handle = pltpu.async_copy(
        pos_hbm.at[pl.ds(pos_base, POS_PER_W)], pos_v, sem_pos)
    for h in idx_handles:
        h.wait()

    g_handles = [None] * NBUF
    o_handles = [None] * NBUF
    for c in range(NBUF - 1):
        g_handles[c] = pltpu.async_copy(
            tok_hbm.at[idx_bufs[c]], tok_bufs[c], g_sems[c])

    for c in range(NCHUNKS):
        buf = c % NBUF
        nc = c + NBUF - 1
        if nc < NCHUNKS:
            nbuf = nc % NBUF
            if o_handles[nbuf] is not None:
                o_handles[nbuf].wait()
            g_handles[nbuf] = pltpu.async_copy(
                tok_hbm.at[idx_bufs[nc]], tok_bufs[nbuf], g_sems[nbuf])
        g_handles[buf].wait()
        if c == 0:
            pos_handle.wait()

        half = c % 2
        tv = tok_bufs[buf]

        @plsc.parallel_loop(0, CHUNK)
        def _row(r):
            for j in range(GROUPS):
                sl = pl.ds(j * LANES, LANES)
                tv[r, sl] = tv[r, sl] * SCALE + pos_v[half * CHUNK + r, sl]
        o_handles[buf] = pltpu.async_copy(
            tv, out_hbm.at[pl.ds(chunk_off(c), CHUNK)], o_sems[buf])

    for h in o_handles:
        if h is not None:
            h.wait()


@jax.jit
def _embed(idx_flat, tok_table, pos_table):
    mesh = plsc.VectorSubcoreMesh(core_axis_name="c", subcore_axis_name="s")
    k = functools.partial(
        pl.kernel,
        mesh=mesh,
        out_type=jax.ShapeDtypeStruct((BATCH * SEQ_LEN, D_MODEL), jnp.float32),
        scratch_types=[
            *[pltpu.VMEM((CHUNK,), jnp.int32) for _ in range(NCHUNKS)],
            *[pltpu.VMEM((CHUNK, D_MODEL), jnp.float32) for _ in range(NBUF)],
            pltpu.VMEM((POS_PER_W, D_MODEL), jnp.float32),
            *[pltpu.SemaphoreType.DMA for _ in range(2 + 2 * NBUF)],
        ],
    )(_body)
    return k(idx_flat, tok_table, pos_table)


def kernel(inputs, tok_table, pos_table):
    idx_flat = inputs.astype(jnp.int32).reshape(-1)
    out = _embed(idx_flat, tok_table, pos_table)
    return out.reshape(BATCH, SEQ_LEN, D_MODEL)
